# Initial kernel scaffold; baseline (speedup 1.0000x reference)
import jax
import jax.numpy as jnp
from jax.experimental import pallas as pl


def kernel(x, table):
    raise NotImplementedError("write your pallas kernel here")


# speedup vs baseline: 18.3481x; 18.3481x over previous
import functools

import jax
import jax.numpy as jnp
from jax import lax
from jax.experimental import pallas as pl
from jax.experimental.pallas import tpu as pltpu
from jax.experimental.pallas import tpu_sc as plsc

_D = 64
_L = 200
_C1 = 104
_C2 = _L - _C1
_NBUF = 4
_RUNROLL = 8


@functools.partial(jax.jit, static_argnames=("num_workers",))
def _sc_mean_pool(idx, table, *, num_workers):
    batch = idx.shape[0] * idx.shape[1] // _L
    bpw = batch // num_workers
    ipw = bpw * _L
    mesh = plsc.VectorSubcoreMesh(core_axis_name="c", subcore_axis_name="s")
    num_cores = mesh.num_cores

    @functools.partial(
        pl.kernel,
        out_type=jax.ShapeDtypeStruct((batch, _D), jnp.float32),
        mesh=mesh,
        scratch_types=[
            pltpu.VMEM((ipw,), jnp.int32),
            pltpu.VMEM((_NBUF, _L, _D), jnp.float32),
            pltpu.VMEM((bpw, _D), jnp.float32),
            pltpu.SemaphoreType.DMA,
        ],
        compiler_params=pltpu.CompilerParams(use_tc_tiling_on_sc=False),
    )
    def body(idx_hbm, table_hbm, out_hbm, idx_v, rows_v, out_v, sem):
        wid = lax.axis_index("s") * num_cores + lax.axis_index("c")
        pltpu.sync_copy(idx_hbm.at[wid], idx_v)

        def row_copies(i, b):
            off = i * _L
            c1 = pltpu.make_async_copy(
                table_hbm.at[idx_v.at[pl.ds(off, _C1)]],
                rows_v.at[b, pl.ds(0, _C1)],
                sem,
            )
            c2 = pltpu.make_async_copy(
                table_hbm.at[idx_v.at[pl.ds(off + _C1, _C2)]],
                rows_v.at[b, pl.ds(_C1, _C2)],
                sem,
            )
            return c1, c2

        def start_row(i, b):
            c1, c2 = row_copies(i, b)
            c1.start()
            c2.start()

        def wait_row(i, b):
            c1, c2 = row_copies(i, b)
            c1.wait()
            c2.wait()

        inv = jnp.float32(1.0 / _L)

        def reduce_row(i, b):
            def rbody(r, accs):
                res = list(accs)
                base = r * _RUNROLL
                for k in range(_RUNROLL):
                    for c in range(_D // 16):
                        res[c] = res[c] + rows_v[b, base + k, pl.ds(c * 16, 16)]
                return tuple(res)

            zero = jnp.zeros((16,), jnp.float32)
            accs = lax.fori_loop(0, _L // _RUNROLL, rbody, (zero,) * (_D // 16))
            for c in range(_D // 16):
                out_v[i, pl.ds(c * 16, 16)] = accs[c] * inv

        for b in range(_NBUF):
            start_row(b, b)

        def group(g, carry):
            for b in range(_NBUF):
                i = g * _NBUF + b
                wait_row(i, b)
                reduce_row(i, b)

                @pl.when(i + _NBUF < bpw)
                def _():
                    start_row(i + _NBUF, b)

            return carry

        lax.fori_loop(0, bpw // _NBUF, group, 0)
        pltpu.sync_copy(out_v, out_hbm.at[pl.ds(wid * bpw, bpw)])

    return body(idx, table)


def kernel(x, table):
    info = plsc.get_sparse_core_info()
    num_workers = info.num_cores * info.num_subcores
    batch = x.shape[0]
    idx = x.astype(jnp.int32).reshape(num_workers, (batch // num_workers) * _L)
    return _sc_mean_pool(idx, table, num_workers=num_workers)
